# baseline (speedup 1.0000x reference)
import functools

import jax
import jax.numpy as jnp
from jax import lax
from jax.experimental import pallas as pl
from jax.experimental.pallas import tpu as pltpu
from jax.experimental.pallas import tpu_sc as plsc

_NC = 2
_NS = 16
_NW = _NC * _NS
_WIN = 80
_NBUF = 3


def _sc_aggregate(h, src3, dst3, zeros):
    n, d = h.shape
    n_win = src3.shape[1]
    _CHUNK = 16
    n_chunks = n // _CHUNK
    mesh = plsc.VectorSubcoreMesh(core_axis_name="c", subcore_axis_name="s")

    @functools.partial(
        pl.kernel,
        mesh=mesh,
        out_type=jax.ShapeDtypeStruct((_NC, n, d), jnp.float32),
        scratch_types=[
            pltpu.VMEM((n_win, _WIN), jnp.int32),
            pltpu.VMEM_SHARED((n, d), jnp.float32),
        ],
    )
    def k(h_hbm, src_hbm, dst_hbm, z_hbm, out_hbm, sidx_all, acc_sh):
        def scoped(*ring):
            rows = ring[:_NBUF]
            didx = ring[_NBUF:2 * _NBUF]
            gsems = ring[2 * _NBUF:3 * _NBUF]
            dsems = ring[3 * _NBUF:4 * _NBUF]
            ssems = ring[4 * _NBUF:]
            _run(h_hbm, src_hbm, dst_hbm, z_hbm, out_hbm,
                 sidx_all, acc_sh, rows, didx, gsems, dsems, ssems)

        pl.run_scoped(scoped,
                      *([pltpu.VMEM((_WIN, d), jnp.float32)] * _NBUF),
                      *([pltpu.VMEM((_WIN,), jnp.int32)] * _NBUF),
                      *([pltpu.SemaphoreType.DMA] * (3 * _NBUF)))

    def _run(h_hbm, src_hbm, dst_hbm, z_hbm, out_hbm,
             sidx_all, acc_sh, rows, didx, gsems, dsems, ssems):
        c = lax.axis_index("c")
        s = lax.axis_index("s")
        wid = c * _NS + s
        dst_my = dst_hbm.at[wid]

        pltpu.sync_copy(src_hbm.at[wid], sidx_all)

        for b in range(_NBUF - 1):
            pltpu.async_copy(h_hbm.at[sidx_all.at[b]], rows[b], gsems[b])
            pltpu.async_copy(dst_my.at[b], didx[b], dsems[b])

        @pl.loop(s, n_chunks, step=_NS)
        def _(ch):
            pltpu.sync_copy(z_hbm.at[pl.ds(ch * _CHUNK, _CHUNK)],
                            acc_sh.at[pl.ds(ch * _CHUNK, _CHUNK)])
        plsc.subcore_barrier()

        @pl.loop(0, n_win, step=_NBUF)
        def _(g):
            for b in range(_NBUF):
                w = g + b
                b2 = (b + 2) % _NBUF

                @pl.when(w < n_win)
                def _():
                    pltpu.make_async_copy(
                        h_hbm.at[pl.ds(0, _WIN)], rows[b], gsems[b]).wait()
                    pltpu.make_async_copy(
                        dst_my.at[w], didx[b], dsems[b]).wait()
                    pltpu.async_copy(rows[b], acc_sh.at[didx[b]], ssems[b],
                                     add=True)

                    @pl.when(w >= 1)
                    def _():
                        pltpu.make_async_copy(
                            rows[b2], acc_sh.at[pl.ds(0, _WIN)],
                            ssems[b2]).wait()

                    @pl.when(w + 2 < n_win)
                    def _():
                        pltpu.async_copy(h_hbm.at[sidx_all.at[w + 2]],
                                         rows[b2], gsems[b2])
                        pltpu.async_copy(dst_my.at[w + 2], didx[b2],
                                         dsems[b2])

        bl = (n_win - 1) % _NBUF
        pltpu.make_async_copy(
            rows[bl], acc_sh.at[pl.ds(0, _WIN)], ssems[bl]).wait()
        plsc.subcore_barrier()

        @pl.loop(s, n_chunks, step=_NS)
        def _(ch):
            pltpu.sync_copy(acc_sh.at[pl.ds(ch * _CHUNK, _CHUNK)],
                            out_hbm.at[c].at[pl.ds(ch * _CHUNK, _CHUNK)])

    return k(h, src3, dst3, zeros)


_BLK = 1000


def _tc_matmul(x, w):
    n, d_in = x.shape
    d_out = w.shape[0]

    def body(x_ref, w_ref, o_ref):
        o_ref[...] = lax.dot_general(
            x_ref[...], w_ref[...], (((1,), (1,)), ((), ())),
            preferred_element_type=jnp.float32)

    return pl.pallas_call(
        body,
        grid=(n // _BLK,),
        in_specs=[
            pl.BlockSpec((_BLK, d_in), lambda i: (i, 0)),
            pl.BlockSpec((d_out, d_in), lambda i: (0, 0)),
        ],
        out_specs=pl.BlockSpec((_BLK, d_out), lambda i: (i, 0)),
        out_shape=jax.ShapeDtypeStruct((n, d_out), jnp.float32),
    )(x, w)


def _tc_combine_matmul(p, b, w):
    _, n, d = p.shape
    d_out = w.shape[0]

    def body(p_ref, b_ref, w_ref, o_ref):
        a = jnp.maximum(p_ref[0] + p_ref[1] + b_ref[...], 0.0)
        o_ref[...] = lax.dot_general(
            a, w_ref[...], (((1,), (1,)), ((), ())),
            preferred_element_type=jnp.float32)

    return pl.pallas_call(
        body,
        grid=(n // _BLK,),
        in_specs=[
            pl.BlockSpec((2, _BLK, d), lambda i: (0, i, 0)),
            pl.BlockSpec((1, d), lambda i: (0, 0)),
            pl.BlockSpec((d_out, d), lambda i: (0, 0)),
        ],
        out_specs=pl.BlockSpec((_BLK, d_out), lambda i: (i, 0)),
        out_shape=jax.ShapeDtypeStruct((n, d_out), jnp.float32),
    )(p, b, w)


def _tc_combine(p, b):
    _, n, d = p.shape

    def body(p_ref, b_ref, o_ref):
        o_ref[...] = jnp.maximum(p_ref[0] + p_ref[1] + b_ref[...], 0.0)

    return pl.pallas_call(
        body,
        grid=(n // _BLK,),
        in_specs=[
            pl.BlockSpec((2, _BLK, d), lambda i: (0, i, 0)),
            pl.BlockSpec((1, d), lambda i: (0, 0)),
        ],
        out_specs=pl.BlockSpec((_BLK, d), lambda i: (i, 0)),
        out_shape=jax.ShapeDtypeStruct((n, d), jnp.float32),
    )(p, b)


def kernel(node_states, edge_index, W1, b1, W2, b2):
    e = edge_index.shape[1]
    n_win = e // (_NW * _WIN)
    src3 = edge_index[0].astype(jnp.int32).reshape(_NW, n_win, _WIN)
    dst3 = edge_index[1].astype(jnp.int32).reshape(_NW, n_win, _WIN)
    n = node_states.shape[0]
    zeros = jnp.zeros((n, W1.shape[0]), jnp.float32)

    h1 = _tc_matmul(node_states, W1)
    p1 = _sc_aggregate(h1, src3, dst3, zeros)
    h2 = _tc_combine_matmul(p1, b1.reshape(1, -1), W2)
    p2 = _sc_aggregate(h2, src3, dst3, zeros)
    return _tc_combine(p2, b2.reshape(1, -1))


# speedup vs baseline: 1.0499x; 1.0499x over previous
import functools

import jax
import jax.numpy as jnp
from jax import lax
from jax.experimental import pallas as pl
from jax.experimental.pallas import tpu as pltpu
from jax.experimental.pallas import tpu_sc as plsc

_NC = 2
_NS = 16
_NW = _NC * _NS
_WIN = 80
_NBUF = 4
_SN = 2 * _NBUF


def _sc_aggregate(h, src3, dst3, zeros):
    n, d = h.shape
    n_win = src3.shape[1]
    _CHUNK = 16
    n_chunks = n // _CHUNK
    mesh = plsc.VectorSubcoreMesh(core_axis_name="c", subcore_axis_name="s")

    @functools.partial(
        pl.kernel,
        mesh=mesh,
        out_type=jax.ShapeDtypeStruct((_NC, n, d), jnp.float32),
        scratch_types=[
            pltpu.VMEM_SHARED((n, d), jnp.float32),
        ],
    )
    def k(h_hbm, src_hbm, dst_hbm, z_hbm, out_hbm, acc_sh):
        def scoped(*ring):
            rows = ring[:_NBUF]
            didx = ring[_NBUF:2 * _NBUF]
            sidx = ring[2 * _NBUF:2 * _NBUF + _SN]
            gsems = ring[2 * _NBUF + _SN:3 * _NBUF + _SN]
            dsems = ring[3 * _NBUF + _SN:4 * _NBUF + _SN]
            isems = ring[4 * _NBUF + _SN:]
            _run(h_hbm, src_hbm, dst_hbm, z_hbm, out_hbm,
                 acc_sh, rows, didx, sidx, gsems, dsems, isems)

        pl.run_scoped(scoped,
                      *([pltpu.VMEM((_WIN, d), jnp.float32)] * _NBUF),
                      *([pltpu.VMEM((_WIN,), jnp.int32)] * _NBUF),
                      *([pltpu.VMEM((_WIN,), jnp.int32)] * _SN),
                      *([pltpu.SemaphoreType.DMA] * (2 * _NBUF + _SN)))

    def _run(h_hbm, src_hbm, dst_hbm, z_hbm, out_hbm,
             acc_sh, rows, didx, sidx, gsems, dsems, isems):
        c = lax.axis_index("c")
        s = lax.axis_index("s")
        wid = c * _NS + s
        dst_my = dst_hbm.at[wid]
        src_my = src_hbm.at[wid]

        for j in range(_SN):
            pltpu.async_copy(src_my.at[j], sidx[j], isems[j])
        for b in range(_NBUF):
            pltpu.make_async_copy(src_my.at[b], sidx[b], isems[b]).wait()
            pltpu.async_copy(h_hbm.at[sidx[b]], rows[b], gsems[b])
            pltpu.async_copy(dst_my.at[b], didx[b], dsems[b])

        @pl.loop(s, n_chunks, step=_NS)
        def _(ch):
            pltpu.sync_copy(z_hbm.at[pl.ds(ch * _CHUNK, _CHUNK)],
                            acc_sh.at[pl.ds(ch * _CHUNK, _CHUNK)])
        plsc.subcore_barrier()

        @pl.loop(0, n_win, step=_SN)
        def _(g):
            for b8 in range(_SN):
                w = g + b8
                b = b8 % _NBUF
                b4 = (b8 + _NBUF) % _SN

                @pl.when(w < n_win)
                def _():
                    pltpu.make_async_copy(
                        h_hbm.at[pl.ds(0, _WIN)], rows[b], gsems[b]).wait()
                    pltpu.make_async_copy(
                        dst_my.at[w], didx[b], dsems[b]).wait()
                    pltpu.sync_copy(rows[b], acc_sh.at[didx[b]], add=True)

                    @pl.when(w + _NBUF < n_win)
                    def _():
                        pltpu.make_async_copy(
                            src_my.at[w + _NBUF], sidx[b4], isems[b4]).wait()
                        pltpu.async_copy(h_hbm.at[sidx[b4]], rows[b],
                                         gsems[b])
                        pltpu.async_copy(dst_my.at[w + _NBUF],
                                         didx[b], dsems[b])

                    @pl.when(w + _SN < n_win)
                    def _():
                        pltpu.async_copy(src_my.at[w + _SN], sidx[b8],
                                         isems[b8])

        plsc.subcore_barrier()

        @pl.loop(s, n_chunks, step=_NS)
        def _(ch):
            pltpu.sync_copy(acc_sh.at[pl.ds(ch * _CHUNK, _CHUNK)],
                            out_hbm.at[c].at[pl.ds(ch * _CHUNK, _CHUNK)])

    return k(h, src3, dst3, zeros)


_BLK = 1000


def _tc_matmul(x, w):
    n, d_in = x.shape
    d_out = w.shape[0]

    def body(x_ref, w_ref, o_ref):
        o_ref[...] = lax.dot_general(
            x_ref[...], w_ref[...], (((1,), (1,)), ((), ())),
            preferred_element_type=jnp.float32)

    return pl.pallas_call(
        body,
        grid=(n // _BLK,),
        in_specs=[
            pl.BlockSpec((_BLK, d_in), lambda i: (i, 0)),
            pl.BlockSpec((d_out, d_in), lambda i: (0, 0)),
        ],
        out_specs=pl.BlockSpec((_BLK, d_out), lambda i: (i, 0)),
        out_shape=jax.ShapeDtypeStruct((n, d_out), jnp.float32),
    )(x, w)


def _tc_combine_matmul(p, b, w):
    _, n, d = p.shape
    d_out = w.shape[0]

    def body(p_ref, b_ref, w_ref, o_ref):
        a = jnp.maximum(p_ref[0] + p_ref[1] + b_ref[...], 0.0)
        o_ref[...] = lax.dot_general(
            a, w_ref[...], (((1,), (1,)), ((), ())),
            preferred_element_type=jnp.float32)

    return pl.pallas_call(
        body,
        grid=(n // _BLK,),
        in_specs=[
            pl.BlockSpec((2, _BLK, d), lambda i: (0, i, 0)),
            pl.BlockSpec((1, d), lambda i: (0, 0)),
            pl.BlockSpec((d_out, d), lambda i: (0, 0)),
        ],
        out_specs=pl.BlockSpec((_BLK, d_out), lambda i: (i, 0)),
        out_shape=jax.ShapeDtypeStruct((n, d_out), jnp.float32),
    )(p, b, w)


def _tc_combine(p, b):
    _, n, d = p.shape

    def body(p_ref, b_ref, o_ref):
        o_ref[...] = jnp.maximum(p_ref[0] + p_ref[1] + b_ref[...], 0.0)

    return pl.pallas_call(
        body,
        grid=(n // _BLK,),
        in_specs=[
            pl.BlockSpec((2, _BLK, d), lambda i: (0, i, 0)),
            pl.BlockSpec((1, d), lambda i: (0, 0)),
        ],
        out_specs=pl.BlockSpec((_BLK, d), lambda i: (i, 0)),
        out_shape=jax.ShapeDtypeStruct((n, d), jnp.float32),
    )(p, b)


def kernel(node_states, edge_index, W1, b1, W2, b2):
    e = edge_index.shape[1]
    n_win = e // (_NW * _WIN)
    src3 = edge_index[0].astype(jnp.int32).reshape(_NW, n_win, _WIN)
    dst3 = edge_index[1].astype(jnp.int32).reshape(_NW, n_win, _WIN)
    n = node_states.shape[0]
    zeros = jnp.zeros((n, W1.shape[0]), jnp.float32)

    h1 = _tc_matmul(node_states, W1)
    p1 = _sc_aggregate(h1, src3, dst3, zeros)
    h2 = _tc_combine_matmul(p1, b1.reshape(1, -1), W2)
    p2 = _sc_aggregate(h2, src3, dst3, zeros)
    return _tc_combine(p2, b2.reshape(1, -1))


# speedup vs baseline: 1.0741x; 1.0230x over previous
import functools

import jax
import jax.numpy as jnp
from jax import lax
from jax.experimental import pallas as pl
from jax.experimental.pallas import tpu as pltpu
from jax.experimental.pallas import tpu_sc as plsc

_NC = 2
_NS = 16
_NW = _NC * _NS
_WIN = 80
_NBUF = 4
_SN = 2 * _NBUF


def _sc_aggregate(h, src3, dst3, zeros):
    n, d = h.shape
    n_win = src3.shape[1]
    _CHUNK = 16
    n_chunks = n // _CHUNK
    mesh = plsc.VectorSubcoreMesh(core_axis_name="c", subcore_axis_name="s")

    @functools.partial(
        pl.kernel,
        mesh=mesh,
        out_type=jax.ShapeDtypeStruct((_NC, n, d), jnp.float32),
        scratch_types=[
            pltpu.VMEM_SHARED((n, d), jnp.float32),
        ],
    )
    def k(h_hbm, src_hbm, dst_hbm, z_hbm, out_hbm, acc_sh):
        def scoped(*ring):
            rows = ring[:_NBUF]
            didx = ring[_NBUF:2 * _NBUF]
            sidx = ring[2 * _NBUF:2 * _NBUF + _SN]
            gsems = ring[2 * _NBUF + _SN:3 * _NBUF + _SN]
            dsems = ring[3 * _NBUF + _SN:4 * _NBUF + _SN]
            isems = ring[4 * _NBUF + _SN:]
            _run(h_hbm, src_hbm, dst_hbm, z_hbm, out_hbm,
                 acc_sh, rows, didx, sidx, gsems, dsems, isems)

        pl.run_scoped(scoped,
                      *([pltpu.VMEM((_WIN, d), jnp.float32)] * _NBUF),
                      *([pltpu.VMEM((_WIN,), jnp.int32)] * _NBUF),
                      *([pltpu.VMEM((_WIN,), jnp.int32)] * _SN),
                      *([pltpu.SemaphoreType.DMA] * (2 * _NBUF + _SN)))

    def _run(h_hbm, src_hbm, dst_hbm, z_hbm, out_hbm,
             acc_sh, rows, didx, sidx, gsems, dsems, isems):
        c = lax.axis_index("c")
        s = lax.axis_index("s")
        wid = c * _NS + s
        dst_my = dst_hbm.at[wid]
        src_my = src_hbm.at[wid]

        for j in range(_SN):
            pltpu.async_copy(src_my.at[j], sidx[j], isems[j])
        for b in range(_NBUF):
            pltpu.make_async_copy(src_my.at[b], sidx[b], isems[b]).wait()
            pltpu.async_copy(h_hbm.at[sidx[b]], rows[b], gsems[b])
            pltpu.async_copy(dst_my.at[b], didx[b], dsems[b])

        @pl.loop(s, n_chunks, step=_NS)
        def _(ch):
            pltpu.sync_copy(z_hbm.at[pl.ds(ch * _CHUNK, _CHUNK)],
                            acc_sh.at[pl.ds(ch * _CHUNK, _CHUNK)])
        plsc.subcore_barrier()

        @pl.loop(0, n_win, step=_SN)
        def _(g):
            for b8 in range(_SN):
                w = g + b8
                b = b8 % _NBUF
                b4 = (b8 + _NBUF) % _SN

                @pl.when(w < n_win)
                def _():
                    pltpu.make_async_copy(
                        h_hbm.at[pl.ds(0, _WIN)], rows[b], gsems[b]).wait()
                    pltpu.make_async_copy(
                        dst_my.at[w], didx[b], dsems[b]).wait()
                    pltpu.sync_copy(rows[b], acc_sh.at[didx[b]], add=True)

                    @pl.when(w + _NBUF < n_win)
                    def _():
                        pltpu.make_async_copy(
                            src_my.at[w + _NBUF], sidx[b4], isems[b4]).wait()
                        pltpu.async_copy(h_hbm.at[sidx[b4]], rows[b],
                                         gsems[b])
                        pltpu.async_copy(dst_my.at[w + _NBUF],
                                         didx[b], dsems[b])

                    @pl.when(w + _SN < n_win)
                    def _():
                        pltpu.async_copy(src_my.at[w + _SN], sidx[b8],
                                         isems[b8])

        plsc.subcore_barrier()

        @pl.loop(s, n_chunks, step=_NS)
        def _(ch):
            pltpu.sync_copy(acc_sh.at[pl.ds(ch * _CHUNK, _CHUNK)],
                            out_hbm.at[c].at[pl.ds(ch * _CHUNK, _CHUNK)])

    return k(h, src3, dst3, zeros)


_BLK = 2000


def _tc_matmul(x, w):
    n, d_in = x.shape
    d_out = w.shape[0]

    def body(x_ref, w_ref, o_ref):
        o_ref[...] = lax.dot_general(
            x_ref[...], w_ref[...], (((1,), (1,)), ((), ())),
            preferred_element_type=jnp.float32)

    return pl.pallas_call(
        body,
        grid=(n // _BLK,),
        in_specs=[
            pl.BlockSpec((_BLK, d_in), lambda i: (i, 0)),
            pl.BlockSpec((d_out, d_in), lambda i: (0, 0)),
        ],
        out_specs=pl.BlockSpec((_BLK, d_out), lambda i: (i, 0)),
        out_shape=jax.ShapeDtypeStruct((n, d_out), jnp.float32),
    )(x, w)


def _tc_combine_matmul(p, b, w):
    _, n, d = p.shape
    d_out = w.shape[0]

    def body(p_ref, b_ref, w_ref, o_ref):
        a = jnp.maximum(p_ref[0] + p_ref[1] + b_ref[...], 0.0)
        o_ref[...] = lax.dot_general(
            a, w_ref[...], (((1,), (1,)), ((), ())),
            preferred_element_type=jnp.float32)

    return pl.pallas_call(
        body,
        grid=(n // _BLK,),
        in_specs=[
            pl.BlockSpec((2, _BLK, d), lambda i: (0, i, 0)),
            pl.BlockSpec((1, d), lambda i: (0, 0)),
            pl.BlockSpec((d_out, d), lambda i: (0, 0)),
        ],
        out_specs=pl.BlockSpec((_BLK, d_out), lambda i: (i, 0)),
        out_shape=jax.ShapeDtypeStruct((n, d_out), jnp.float32),
    )(p, b, w)


def _tc_combine(p, b):
    _, n, d = p.shape

    def body(p_ref, b_ref, o_ref):
        o_ref[...] = jnp.maximum(p_ref[0] + p_ref[1] + b_ref[...], 0.0)

    return pl.pallas_call(
        body,
        grid=(n // _BLK,),
        in_specs=[
            pl.BlockSpec((2, _BLK, d), lambda i: (0, i, 0)),
            pl.BlockSpec((1, d), lambda i: (0, 0)),
        ],
        out_specs=pl.BlockSpec((_BLK, d), lambda i: (i, 0)),
        out_shape=jax.ShapeDtypeStruct((n, d), jnp.float32),
    )(p, b)


def kernel(node_states, edge_index, W1, b1, W2, b2):
    e = edge_index.shape[1]
    n_win = e // (_NW * _WIN)
    src3 = edge_index[0].astype(jnp.int32).reshape(_NW, n_win, _WIN)
    dst3 = edge_index[1].astype(jnp.int32).reshape(_NW, n_win, _WIN)
    n = node_states.shape[0]
    zeros = jnp.zeros((n, W1.shape[0]), jnp.float32)

    h1 = _tc_matmul(node_states, W1)
    p1 = _sc_aggregate(h1, src3, dst3, zeros)
    h2 = _tc_combine_matmul(p1, b1.reshape(1, -1), W2)
    p2 = _sc_aggregate(h2, src3, dst3, zeros)
    return _tc_combine(p2, b2.reshape(1, -1))


# speedup vs baseline: 1.0922x; 1.0169x over previous
import functools

import jax
import jax.numpy as jnp
from jax import lax
from jax.experimental import pallas as pl
from jax.experimental.pallas import tpu as pltpu
from jax.experimental.pallas import tpu_sc as plsc

_NC = 2
_NS = 16
_NW = _NC * _NS
_WIN = 80
_NBUF = 4
_SN = 2 * _NBUF


def _sc_aggregate(h, src3, dst3, zeros):
    n, d = h.shape
    n_win = src3.shape[1]
    _CHUNK = 16
    n_chunks = n // _CHUNK
    mesh = plsc.VectorSubcoreMesh(core_axis_name="c", subcore_axis_name="s")

    @functools.partial(
        pl.kernel,
        mesh=mesh,
        out_type=jax.ShapeDtypeStruct((_NC, n, d), jnp.float32),
        scratch_types=[
            pltpu.VMEM_SHARED((n, d), jnp.float32),
        ],
    )
    def k(h_hbm, src_hbm, dst_hbm, z_hbm, out_hbm, acc_sh):
        def scoped(*ring):
            rows = ring[:_NBUF]
            didx = ring[_NBUF:2 * _NBUF]
            sidx = ring[2 * _NBUF:2 * _NBUF + _SN]
            gsems = ring[2 * _NBUF + _SN:3 * _NBUF + _SN]
            dsems = ring[3 * _NBUF + _SN:4 * _NBUF + _SN]
            isems = ring[4 * _NBUF + _SN:]
            _run(h_hbm, src_hbm, dst_hbm, z_hbm, out_hbm,
                 acc_sh, rows, didx, sidx, gsems, dsems, isems)

        pl.run_scoped(scoped,
                      *([pltpu.VMEM((_WIN, d), jnp.float32)] * _NBUF),
                      *([pltpu.VMEM((_WIN,), jnp.int32)] * _NBUF),
                      *([pltpu.VMEM((_WIN,), jnp.int32)] * _SN),
                      *([pltpu.SemaphoreType.DMA] * (2 * _NBUF + _SN)))

    def _run(h_hbm, src_hbm, dst_hbm, z_hbm, out_hbm,
             acc_sh, rows, didx, sidx, gsems, dsems, isems):
        c = lax.axis_index("c")
        s = lax.axis_index("s")
        wid = c * _NS + s
        dst_my = dst_hbm.at[wid]
        src_my = src_hbm.at[wid]

        for j in range(_SN):
            pltpu.async_copy(src_my.at[j], sidx[j], isems[j])
        for b in range(_NBUF):
            pltpu.make_async_copy(src_my.at[b], sidx[b], isems[b]).wait()
            pltpu.async_copy(h_hbm.at[sidx[b]], rows[b], gsems[b])
            pltpu.async_copy(dst_my.at[b], didx[b], dsems[b])

        @pl.loop(s, n_chunks, step=_NS)
        def _(ch):
            pltpu.sync_copy(z_hbm.at[pl.ds(ch * _CHUNK, _CHUNK)],
                            acc_sh.at[pl.ds(ch * _CHUNK, _CHUNK)])
        plsc.subcore_barrier()

        @pl.loop(0, n_win, step=_SN)
        def _(g):
            for b8 in range(_SN):
                w = g + b8
                b = b8 % _NBUF
                b4 = (b8 + _NBUF) % _SN

                @pl.when(w < n_win)
                def _():
                    pltpu.make_async_copy(
                        h_hbm.at[pl.ds(0, _WIN)], rows[b], gsems[b]).wait()
                    pltpu.make_async_copy(
                        dst_my.at[w], didx[b], dsems[b]).wait()
                    pltpu.sync_copy(rows[b], acc_sh.at[didx[b]], add=True)

                    @pl.when(w + _NBUF < n_win)
                    def _():
                        pltpu.make_async_copy(
                            src_my.at[w + _NBUF], sidx[b4], isems[b4]).wait()
                        pltpu.async_copy(h_hbm.at[sidx[b4]], rows[b],
                                         gsems[b])
                        pltpu.async_copy(dst_my.at[w + _NBUF],
                                         didx[b], dsems[b])

                    @pl.when(w + _SN < n_win)
                    def _():
                        pltpu.async_copy(src_my.at[w + _SN], sidx[b8],
                                         isems[b8])

        plsc.subcore_barrier()

        @pl.loop(s, n_chunks, step=_NS)
        def _(ch):
            pltpu.sync_copy(acc_sh.at[pl.ds(ch * _CHUNK, _CHUNK)],
                            out_hbm.at[c].at[pl.ds(ch * _CHUNK, _CHUNK)])

    return k(h, src3, dst3, zeros)


_BLK = 5000


def _tc_matmul(x, w):
    n, d_in = x.shape
    d_out = w.shape[0]

    def body(x_ref, w_ref, o_ref):
        o_ref[...] = lax.dot_general(
            x_ref[...], w_ref[...], (((1,), (1,)), ((), ())),
            preferred_element_type=jnp.float32)

    return pl.pallas_call(
        body,
        grid=(n // _BLK,),
        in_specs=[
            pl.BlockSpec((_BLK, d_in), lambda i: (i, 0)),
            pl.BlockSpec((d_out, d_in), lambda i: (0, 0)),
        ],
        out_specs=pl.BlockSpec((_BLK, d_out), lambda i: (i, 0)),
        out_shape=jax.ShapeDtypeStruct((n, d_out), jnp.float32),
    )(x, w)


def _tc_combine_matmul(p, b, w):
    _, n, d = p.shape
    d_out = w.shape[0]

    def body(p_ref, b_ref, w_ref, o_ref):
        a = jnp.maximum(p_ref[0] + p_ref[1] + b_ref[...], 0.0)
        o_ref[...] = lax.dot_general(
            a, w_ref[...], (((1,), (1,)), ((), ())),
            preferred_element_type=jnp.float32)

    return pl.pallas_call(
        body,
        grid=(n // _BLK,),
        in_specs=[
            pl.BlockSpec((2, _BLK, d), lambda i: (0, i, 0)),
            pl.BlockSpec((1, d), lambda i: (0, 0)),
            pl.BlockSpec((d_out, d), lambda i: (0, 0)),
        ],
        out_specs=pl.BlockSpec((_BLK, d_out), lambda i: (i, 0)),
        out_shape=jax.ShapeDtypeStruct((n, d_out), jnp.float32),
    )(p, b, w)


def _tc_combine(p, b):
    _, n, d = p.shape

    def body(p_ref, b_ref, o_ref):
        o_ref[...] = jnp.maximum(p_ref[0] + p_ref[1] + b_ref[...], 0.0)

    return pl.pallas_call(
        body,
        grid=(n // _BLK,),
        in_specs=[
            pl.BlockSpec((2, _BLK, d), lambda i: (0, i, 0)),
            pl.BlockSpec((1, d), lambda i: (0, 0)),
        ],
        out_specs=pl.BlockSpec((_BLK, d), lambda i: (i, 0)),
        out_shape=jax.ShapeDtypeStruct((n, d), jnp.float32),
    )(p, b)


def kernel(node_states, edge_index, W1, b1, W2, b2):
    e = edge_index.shape[1]
    n_win = e // (_NW * _WIN)
    src3 = edge_index[0].astype(jnp.int32).reshape(_NW, n_win, _WIN)
    dst3 = edge_index[1].astype(jnp.int32).reshape(_NW, n_win, _WIN)
    n = node_states.shape[0]
    zeros = jnp.zeros((n, W1.shape[0]), jnp.float32)

    h1 = _tc_matmul(node_states, W1)
    p1 = _sc_aggregate(h1, src3, dst3, zeros)
    h2 = _tc_combine_matmul(p1, b1.reshape(1, -1), W2)
    p2 = _sc_aggregate(h2, src3, dst3, zeros)
    return _tc_combine(p2, b2.reshape(1, -1))
